# baseline (speedup 1.0000x reference)
import functools

import jax
import jax.numpy as jnp
import numpy as np
from jax import lax
from jax.experimental import pallas as pl
from jax.experimental.pallas import tpu as pltpu
from jax.experimental.pallas import tpu_sc as plsc

CONF_T = 0.2
IOU_T = 0.7
MAX_DET = 100
STRIDES = (8, 16, 32)
IMG_H, IMG_W = 512, 512
NUM_CLASSES = 80
N = sum((IMG_H // s) * (IMG_W // s) for s in STRIDES)
LANES = 16
NCHUNK = N // LANES
NEG_INF = float("-inf")
BIGC = 1.0e30


def _anchor_meta():
    ax_l, ay_l, st_l = [], [], []
    for s in STRIDES:
        hh = np.arange(0, IMG_H, s, dtype=np.float32)
        ww = np.arange(0, IMG_W, s, dtype=np.float32)
        ww_g, hh_g = np.meshgrid(ww, hh)
        ay = (hh_g.reshape(-1) + 0.5 * s) / s
        ax = (ww_g.reshape(-1) + 0.5 * s) / s
        ax_l.append(ax)
        ay_l.append(ay)
        st_l.append(np.full(ax.shape, s, dtype=np.float32))
    meta = np.zeros((8, N), dtype=np.float32)
    meta[0] = np.concatenate(ax_l)
    meta[1] = np.concatenate(ay_l)
    meta[2] = np.concatenate(st_l)
    return jnp.asarray(meta)


def _decode_body(boxes_ref, classes_ref, meta_ref, x1_ref, y1_ref, x2_ref,
                 y2_ref, ar_ref, wk_ref):
    x = boxes_ref[0]
    db = []
    kcol = lax.broadcasted_iota(jnp.int32, (16, 1), 0).astype(jnp.float32)
    for s in range(4):
        xs = x[16 * s:16 * s + 16, :]
        m = jnp.max(xs, axis=0, keepdims=True)
        e = jnp.exp(xs - m)
        den = jnp.sum(e, axis=0, keepdims=True)
        num = jnp.sum(e * kcol, axis=0, keepdims=True)
        db.append(num / den)
    ax = meta_ref[0:1, :]
    ay = meta_ref[1:2, :]
    st = meta_ref[2:3, :]
    x1 = (ax - db[0]) * st
    y1 = (ay - db[1]) * st
    x2 = (ax + db[2]) * st
    y2 = (ay + db[3]) * st
    x1_ref[0] = x1
    y1_ref[0] = y1
    x2_ref[0] = x2
    y2_ref[0] = y2
    ar_ref[0] = jnp.maximum(x2 - x1, 0.0) * jnp.maximum(y2 - y1, 0.0)
    conf = jnp.max(classes_ref[0], axis=0, keepdims=True)
    wk_ref[0] = jnp.where(conf > CONF_T, conf, NEG_INF)


def _decode(boxes_t, classes_t, meta, batch):
    row = jax.ShapeDtypeStruct((batch, 1, N), jnp.float32)
    return pl.pallas_call(
        _decode_body,
        grid=(batch,),
        in_specs=[
            pl.BlockSpec((1, 64, N), lambda b: (b, 0, 0)),
            pl.BlockSpec((1, NUM_CLASSES, N), lambda b: (b, 0, 0)),
            pl.BlockSpec((8, N), lambda b: (0, 0)),
        ],
        out_specs=[pl.BlockSpec((1, 1, N), lambda b: (b, 0, 0))] * 6,
        out_shape=[row] * 6,
    )(boxes_t, classes_t, meta)


def _nms_body(x1h, y1h, x2h, y2h, arh, wkh, dsh, outh,
              x1v, y1v, x2v, y2v, arv, wkv, d0v, d1v, d2v, d3v, outv,
              scrf, scri):
    info = plsc.get_sparse_core_info()
    nc = info.num_cores
    w = lax.axis_index("s") * nc + lax.axis_index("c")

    @pl.when(w < x1h.shape[0])
    def _():
        pltpu.sync_copy(x1h.at[w, 0], x1v)
        pltpu.sync_copy(y1h.at[w, 0], y1v)
        pltpu.sync_copy(x2h.at[w, 0], x2v)
        pltpu.sync_copy(y2h.at[w, 0], y2v)
        pltpu.sync_copy(arh.at[w, 0], arv)
        pltpu.sync_copy(wkh.at[w, 0], wkv)
        pltpu.sync_copy(dsh.at[w, 0], d0v)
        pltpu.sync_copy(dsh.at[w, 1], d1v)
        pltpu.sync_copy(dsh.at[w, 2], d2v)
        pltpu.sync_copy(dsh.at[w, 3], d3v)

        lane = lax.iota(jnp.int32, 16)
        ninf = jnp.full((16,), NEG_INF, jnp.float32)
        zeroi = jnp.zeros((16,), jnp.int32)

        def init_chunk(c, carry):
            best, bidx = carry
            v = wkv[pl.ds(c * 16, 16)]
            gidx = lane + c * 16
            upd = v > best
            return jnp.where(upd, v, best), jnp.where(upd, gidx, bidx)

        best, bidx = lax.fori_loop(0, NCHUNK, init_chunk, (ninf, zeroi))

        def allmax_f(v):
            for k in (8, 4, 2, 1):
                scrf[...] = v
                v = jnp.maximum(v, plsc.load_gather(scrf, [lane ^ k]))
            return v

        def allmin_i(v):
            for k in (8, 4, 2, 1):
                scri[...] = v
                v = jnp.minimum(v, plsc.load_gather(scri, [lane ^ k]))
            return v

        def det_step(d, carry):
            best, bidx = carry
            m = allmax_f(best)
            isp = allmin_i(
                jnp.where(best == m, bidx, jnp.int32(0x7FFFFFFF)))
            vsp = m > NEG_INF
            x1g = plsc.load_gather(x1v, [isp])
            y1g = plsc.load_gather(y1v, [isp])
            x2g = plsc.load_gather(x2v, [isp])
            y2g = plsc.load_gather(y2v, [isp])
            arg_ = plsc.load_gather(arv, [isp])
            x1s = jnp.where(vsp, x1g, BIGC)
            y1s = jnp.where(vsp, y1g, BIGC)
            x2s = jnp.where(vsp, x2g, -BIGC)
            y2s = jnp.where(vsp, y2g, -BIGC)
            ars = jnp.where(vsp, arg_, 0.0)
            plsc.store_scatter(wkv, [isp], ninf)

            unroll = 4
            step = 16 * unroll

            def sweep_chunk(c, carry2):
                bs = list(carry2[:unroll])
                bis = list(carry2[unroll:])
                base = c * step
                for u in range(unroll):
                    sl = pl.ds(base + u * 16, 16)
                    iw = jnp.maximum(
                        jnp.minimum(x2s, x2v[sl]) - jnp.maximum(x1s, x1v[sl]),
                        0.0)
                    ih = jnp.maximum(
                        jnp.minimum(y2s, y2v[sl]) - jnp.maximum(y1s, y1v[sl]),
                        0.0)
                    inter = iw * ih
                    kill = inter > IOU_T * (ars + arv[sl] - inter)
                    wc = jnp.where(kill, NEG_INF, wkv[sl])
                    wkv[sl] = wc
                    gidx = lane + (base + u * 16)
                    upd = wc > bs[u]
                    bs[u] = jnp.where(upd, wc, bs[u])
                    bis[u] = jnp.where(upd, gidx, bis[u])
                return tuple(bs) + tuple(bis)

            acc = lax.fori_loop(0, NCHUNK // unroll, sweep_chunk,
                                (ninf,) * unroll + (zeroi,) * unroll)
            best2 = acc[0]
            bidx2 = acc[unroll]
            for u in range(1, unroll):
                vu = acc[u]
                iu = acc[unroll + u]
                upd = (vu > best2) | ((vu == best2) & (iu < bidx2))
                best2 = jnp.where(upd, vu, best2)
                bidx2 = jnp.where(upd, iu, bidx2)

            d0s = plsc.load_gather(d0v, [isp])
            d1s = plsc.load_gather(d1v, [isp])
            d2s = plsc.load_gather(d2v, [isp])
            d3s = plsc.load_gather(d3v, [isp])
            row = jnp.zeros((16,), jnp.float32)
            vals = (x1g, y1g, x2g, y2g, m, d0s, d1s, d2s, d3s)
            for j, vv in enumerate(vals):
                row = jnp.where(lane == j, vv, row)
            row = jnp.where(vsp, row, 0.0)
            plsc.store_scatter(outv, [jnp.full((16,), d, jnp.int32), lane],
                               row)
            return best2, bidx2

        lax.fori_loop(0, MAX_DET, det_step, (best, bidx))
        pltpu.sync_copy(outv, outh.at[w])


def _nms(rows, dist_t, batch):
    mesh = plsc.VectorSubcoreMesh(core_axis_name="c", subcore_axis_name="s")
    vec = pltpu.VMEM((N,), jnp.float32)
    f = pl.kernel(
        _nms_body,
        out_type=jax.ShapeDtypeStruct((batch, 112, 16), jnp.float32),
        mesh=mesh,
        compiler_params=pltpu.CompilerParams(needs_layout_passes=False),
        scratch_types=[vec] * 10 + [
            pltpu.VMEM((112, 16), jnp.float32),
            pltpu.VMEM((16,), jnp.float32),
            pltpu.VMEM((16,), jnp.int32),
        ],
    )
    return f(*rows, dist_t)


def kernel(boxes, classes, distances, images):
    del images
    batch = boxes.shape[0]
    meta = _anchor_meta()
    boxes_t = jnp.transpose(boxes, (0, 2, 1))
    classes_t = jnp.transpose(classes, (0, 2, 1))
    dist_t = jnp.transpose(distances, (0, 2, 1))
    rows = _decode(boxes_t, classes_t, meta, batch)
    out = _nms(rows, dist_t, batch)
    return out[:, :MAX_DET, :9]


# speedup vs baseline: 1.0228x; 1.0228x over previous
import functools

import jax
import jax.numpy as jnp
import numpy as np
from jax import lax
from jax.experimental import pallas as pl
from jax.experimental.pallas import tpu as pltpu
from jax.experimental.pallas import tpu_sc as plsc

CONF_T = 0.2
IOU_T = 0.7
MAX_DET = 100
STRIDES = (8, 16, 32)
IMG_H, IMG_W = 512, 512
NUM_CLASSES = 80
N = sum((IMG_H // s) * (IMG_W // s) for s in STRIDES)
LANES = 16
NCHUNK = N // LANES
NEG_INF = float("-inf")
BIGC = 1.0e30
T2 = IOU_T / (1.0 + IOU_T)


def _anchor_meta():
    ax_l, ay_l, st_l = [], [], []
    for s in STRIDES:
        hh = np.arange(0, IMG_H, s, dtype=np.float32)
        ww = np.arange(0, IMG_W, s, dtype=np.float32)
        ww_g, hh_g = np.meshgrid(ww, hh)
        ay = (hh_g.reshape(-1) + 0.5 * s) / s
        ax = (ww_g.reshape(-1) + 0.5 * s) / s
        ax_l.append(ax)
        ay_l.append(ay)
        st_l.append(np.full(ax.shape, s, dtype=np.float32))
    meta = np.zeros((8, N), dtype=np.float32)
    meta[0] = np.concatenate(ax_l)
    meta[1] = np.concatenate(ay_l)
    meta[2] = np.concatenate(st_l)
    return jnp.asarray(meta)


def _decode_body(boxes_ref, classes_ref, meta_ref, x1_ref, y1_ref, x2_ref,
                 y2_ref, ar_ref, wk_ref):
    x = boxes_ref[0]
    db = []
    kcol = lax.broadcasted_iota(jnp.int32, (16, 1), 0).astype(jnp.float32)
    for s in range(4):
        xs = x[16 * s:16 * s + 16, :]
        m = jnp.max(xs, axis=0, keepdims=True)
        e = jnp.exp(xs - m)
        den = jnp.sum(e, axis=0, keepdims=True)
        num = jnp.sum(e * kcol, axis=0, keepdims=True)
        db.append(num / den)
    ax = meta_ref[0:1, :]
    ay = meta_ref[1:2, :]
    st = meta_ref[2:3, :]
    x1 = (ax - db[0]) * st
    y1 = (ay - db[1]) * st
    x2 = (ax + db[2]) * st
    y2 = (ay + db[3]) * st
    x1_ref[0] = x1
    y1_ref[0] = y1
    x2_ref[0] = x2
    y2_ref[0] = y2
    ar_ref[0] = jnp.maximum(x2 - x1, 0.0) * jnp.maximum(y2 - y1, 0.0)
    conf = jnp.max(classes_ref[0], axis=0, keepdims=True)
    wk_ref[0] = jnp.where(conf > CONF_T, conf, NEG_INF)


def _decode(boxes_t, classes_t, meta, batch):
    row = jax.ShapeDtypeStruct((batch, 1, N), jnp.float32)
    return pl.pallas_call(
        _decode_body,
        grid=(batch,),
        in_specs=[
            pl.BlockSpec((1, 64, N), lambda b: (b, 0, 0)),
            pl.BlockSpec((1, NUM_CLASSES, N), lambda b: (b, 0, 0)),
            pl.BlockSpec((8, N), lambda b: (0, 0)),
        ],
        out_specs=[pl.BlockSpec((1, 1, N), lambda b: (b, 0, 0))] * 6,
        out_shape=[row] * 6,
    )(boxes_t, classes_t, meta)


def _nms_body(x1h, y1h, x2h, y2h, arh, wkh, dsh, outh,
              x1v, y1v, x2v, y2v, arv, wkv, d0v, d1v, d2v, d3v, outv,
              scrf, scri, t2v):
    info = plsc.get_sparse_core_info()
    nc = info.num_cores
    w = lax.axis_index("s") * nc + lax.axis_index("c")

    @pl.when(w < x1h.shape[0])
    def _():
        pltpu.sync_copy(x1h.at[w, 0], x1v)
        pltpu.sync_copy(y1h.at[w, 0], y1v)
        pltpu.sync_copy(x2h.at[w, 0], x2v)
        pltpu.sync_copy(y2h.at[w, 0], y2v)
        pltpu.sync_copy(arh.at[w, 0], arv)
        pltpu.sync_copy(wkh.at[w, 0], wkv)
        pltpu.sync_copy(dsh.at[w, 0], d0v)
        pltpu.sync_copy(dsh.at[w, 1], d1v)
        pltpu.sync_copy(dsh.at[w, 2], d2v)
        pltpu.sync_copy(dsh.at[w, 3], d3v)

        lane = lax.iota(jnp.int32, 16)
        ninf = jnp.full((16,), NEG_INF, jnp.float32)
        zeroi = jnp.zeros((16,), jnp.int32)

        def init_chunk(c, carry):
            best, bidx = carry
            v = wkv[pl.ds(c * 16, 16)]
            gidx = lane + c * 16
            upd = v > best
            return jnp.where(upd, v, best), jnp.where(upd, gidx, bidx)

        best, bidx = lax.fori_loop(0, NCHUNK, init_chunk, (ninf, zeroi))

        def t2_chunk(c, carry):
            sl = pl.ds(c * 16, 16)
            t2v[sl] = arv[sl] * T2
            return carry

        lax.fori_loop(0, NCHUNK, t2_chunk, 0)

        def allmax_f(v):
            for k in (8, 4, 2, 1):
                scrf[...] = v
                v = jnp.maximum(v, plsc.load_gather(scrf, [lane ^ k]))
            return v

        def allmin_i(v):
            for k in (8, 4, 2, 1):
                scri[...] = v
                v = jnp.minimum(v, plsc.load_gather(scri, [lane ^ k]))
            return v

        def det_step(d, carry):
            best, bidx = carry
            m = allmax_f(best)
            isp = allmin_i(
                jnp.where(best == m, bidx, jnp.int32(0x7FFFFFFF)))
            vsp = m > NEG_INF
            x1g = plsc.load_gather(x1v, [isp])
            y1g = plsc.load_gather(y1v, [isp])
            x2g = plsc.load_gather(x2v, [isp])
            y2g = plsc.load_gather(y2v, [isp])
            arg_ = plsc.load_gather(arv, [isp])
            x1s = jnp.where(vsp, x1g, BIGC)
            y1s = jnp.where(vsp, y1g, BIGC)
            x2s = jnp.where(vsp, x2g, -BIGC)
            y2s = jnp.where(vsp, y2g, -BIGC)
            ars = jnp.where(vsp, arg_, 0.0)
            plsc.store_scatter(wkv, [isp], ninf)

            t2sel = ars * T2
            unroll = 4
            step = 16 * unroll

            def sweep_chunk(c, carry2):
                b2, bi2 = carry2
                base = c * step
                for u in range(unroll):
                    sl = pl.ds(base + u * 16, 16)
                    iw = jnp.maximum(
                        jnp.minimum(x2s, x2v[sl]) - jnp.maximum(x1s, x1v[sl]),
                        0.0)
                    ih = jnp.maximum(
                        jnp.minimum(y2s, y2v[sl]) - jnp.maximum(y1s, y1v[sl]),
                        0.0)
                    inter = iw * ih
                    kill = inter > t2sel + t2v[sl]
                    wc = jnp.where(kill, NEG_INF, wkv[sl])
                    wkv[sl] = wc
                    gidx = lane + (base + u * 16)
                    upd = wc > b2
                    b2 = jnp.where(upd, wc, b2)
                    bi2 = jnp.where(upd, gidx, bi2)
                return b2, bi2

            best2, bidx2 = lax.fori_loop(0, NCHUNK // unroll, sweep_chunk,
                                         (ninf, zeroi))

            d0s = plsc.load_gather(d0v, [isp])
            d1s = plsc.load_gather(d1v, [isp])
            d2s = plsc.load_gather(d2v, [isp])
            d3s = plsc.load_gather(d3v, [isp])
            row = jnp.zeros((16,), jnp.float32)
            vals = (x1g, y1g, x2g, y2g, m, d0s, d1s, d2s, d3s)
            for j, vv in enumerate(vals):
                row = jnp.where(lane == j, vv, row)
            row = jnp.where(vsp, row, 0.0)
            plsc.store_scatter(outv, [jnp.full((16,), d, jnp.int32), lane],
                               row)
            return best2, bidx2

        lax.fori_loop(0, MAX_DET, det_step, (best, bidx))
        pltpu.sync_copy(outv, outh.at[w])


def _nms(rows, dist_t, batch):
    mesh = plsc.VectorSubcoreMesh(core_axis_name="c", subcore_axis_name="s")
    vec = pltpu.VMEM((N,), jnp.float32)
    f = pl.kernel(
        _nms_body,
        out_type=jax.ShapeDtypeStruct((batch, 112, 16), jnp.float32),
        mesh=mesh,
        compiler_params=pltpu.CompilerParams(needs_layout_passes=False),
        scratch_types=[vec] * 10 + [
            pltpu.VMEM((112, 16), jnp.float32),
            pltpu.VMEM((16,), jnp.float32),
            pltpu.VMEM((16,), jnp.int32),
            vec,
        ],
    )
    return f(*rows, dist_t)


def kernel(boxes, classes, distances, images):
    del images
    batch = boxes.shape[0]
    meta = _anchor_meta()
    boxes_t = jnp.transpose(boxes, (0, 2, 1))
    classes_t = jnp.transpose(classes, (0, 2, 1))
    dist_t = jnp.transpose(distances, (0, 2, 1))
    rows = _decode(boxes_t, classes_t, meta, batch)
    out = _nms(rows, dist_t, batch)
    return out[:, :MAX_DET, :9]


# speedup vs baseline: 1.0284x; 1.0054x over previous
import functools

import jax
import jax.numpy as jnp
import numpy as np
from jax import lax
from jax.experimental import pallas as pl
from jax.experimental.pallas import tpu as pltpu
from jax.experimental.pallas import tpu_sc as plsc

CONF_T = 0.2
IOU_T = 0.7
MAX_DET = 100
STRIDES = (8, 16, 32)
IMG_H, IMG_W = 512, 512
NUM_CLASSES = 80
N = sum((IMG_H // s) * (IMG_W // s) for s in STRIDES)
LANES = 16
NCHUNK = N // LANES
QUARTER = N // 4
QCHUNK = QUARTER // LANES
NEG_INF = float("-inf")
BIGC = 1.0e30
T2 = IOU_T / (1.0 + IOU_T)


def _anchor_meta():
    ax_l, ay_l, st_l = [], [], []
    for s in STRIDES:
        hh = np.arange(0, IMG_H, s, dtype=np.float32)
        ww = np.arange(0, IMG_W, s, dtype=np.float32)
        ww_g, hh_g = np.meshgrid(ww, hh)
        ay = (hh_g.reshape(-1) + 0.5 * s) / s
        ax = (ww_g.reshape(-1) + 0.5 * s) / s
        ax_l.append(ax)
        ay_l.append(ay)
        st_l.append(np.full(ax.shape, s, dtype=np.float32))
    meta = np.zeros((8, N), dtype=np.float32)
    meta[0] = np.concatenate(ax_l)
    meta[1] = np.concatenate(ay_l)
    meta[2] = np.concatenate(st_l)
    return jnp.asarray(meta)


def _decode_body(boxes_ref, classes_ref, meta_ref, x1_ref, y1_ref, x2_ref,
                 y2_ref, ar_ref, wk_ref):
    x = boxes_ref[0]
    db = []
    kcol = lax.broadcasted_iota(jnp.int32, (16, 1), 0).astype(jnp.float32)
    for s in range(4):
        xs = x[16 * s:16 * s + 16, :]
        m = jnp.max(xs, axis=0, keepdims=True)
        e = jnp.exp(xs - m)
        den = jnp.sum(e, axis=0, keepdims=True)
        num = jnp.sum(e * kcol, axis=0, keepdims=True)
        db.append(num / den)
    ax = meta_ref[0:1, :]
    ay = meta_ref[1:2, :]
    st = meta_ref[2:3, :]
    x1 = (ax - db[0]) * st
    y1 = (ay - db[1]) * st
    x2 = (ax + db[2]) * st
    y2 = (ay + db[3]) * st
    x1_ref[0] = x1
    y1_ref[0] = y1
    x2_ref[0] = x2
    y2_ref[0] = y2
    ar_ref[0] = jnp.maximum(x2 - x1, 0.0) * jnp.maximum(y2 - y1, 0.0)
    conf = jnp.max(classes_ref[0], axis=0, keepdims=True)
    wk_ref[0] = jnp.where(conf > CONF_T, conf, NEG_INF)


def _decode(boxes_t, classes_t, meta, batch):
    row = jax.ShapeDtypeStruct((batch, 1, N), jnp.float32)
    return pl.pallas_call(
        _decode_body,
        grid=(batch,),
        in_specs=[
            pl.BlockSpec((1, 64, N), lambda b: (b, 0, 0)),
            pl.BlockSpec((1, NUM_CLASSES, N), lambda b: (b, 0, 0)),
            pl.BlockSpec((8, N), lambda b: (0, 0)),
        ],
        out_specs=[pl.BlockSpec((1, 1, N), lambda b: (b, 0, 0))] * 6,
        out_shape=[row] * 6,
    )(boxes_t, classes_t, meta)


def _nms_body(x1h, y1h, x2h, y2h, arh, wkh, dsh, outh,
              x1v, y1v, x2v, y2v, arv, wkv, d0v, d1v, d2v, d3v, outv,
              scrf, scri, scrj, t2v, smv):
    sub = lax.axis_index("s")
    core = lax.axis_index("c")
    b = core * 4 + sub // 4
    q = sub % 4
    qbase = q * QUARTER
    g0 = (sub // 4) * 4

    pltpu.sync_copy(x1h.at[b, 0], x1v)
    pltpu.sync_copy(y1h.at[b, 0], y1v)
    pltpu.sync_copy(x2h.at[b, 0], x2v)
    pltpu.sync_copy(y2h.at[b, 0], y2v)
    pltpu.sync_copy(arh.at[b, 0], arv)
    pltpu.sync_copy(wkh.at[b, 0], wkv)

    @pl.when(q == 0)
    def _():
        pltpu.sync_copy(dsh.at[b, 0], d0v)
        pltpu.sync_copy(dsh.at[b, 1], d1v)
        pltpu.sync_copy(dsh.at[b, 2], d2v)
        pltpu.sync_copy(dsh.at[b, 3], d3v)

    lane = lax.iota(jnp.int32, 16)
    ninf = jnp.full((16,), NEG_INF, jnp.float32)
    zeroi = jnp.zeros((16,), jnp.int32)

    def init_chunk(c, carry):
        best, bidx = carry
        v = wkv[pl.ds(qbase + c * 16, 16)]
        gidx = lane + (qbase + c * 16)
        upd = v > best
        return jnp.where(upd, v, best), jnp.where(upd, gidx, bidx)

    best, bidx = lax.fori_loop(0, QCHUNK, init_chunk, (ninf, zeroi))

    def t2_chunk(c, carry):
        sl = pl.ds(qbase + c * 16, 16)
        t2v[sl] = arv[sl] * T2
        return carry

    lax.fori_loop(0, QCHUNK, t2_chunk, 0)

    def allmax_f(v):
        for k in (8, 4, 2, 1):
            scrf[...] = v
            v = jnp.maximum(v, plsc.load_gather(scrf, [lane ^ k]))
        return v

    def allmin_i(v):
        for k in (8, 4, 2, 1):
            scri[...] = v
            v = jnp.minimum(v, plsc.load_gather(scri, [lane ^ k]))
        return v

    def det_step(d, carry):
        best, bidx = carry
        mloc = allmax_f(best)
        iloc = allmin_i(jnp.where(best == mloc, bidx, jnp.int32(0x7FFFFFFF)))
        kb = plsc.bitcast(mloc, jnp.int32)[0]
        il = iloc[0]
        smv[0] = kb
        smv[1] = il
        plsc.subcore_barrier()
        kv = jnp.full((16,), plsc.fetch_and_add(smv.at[0], 0, subcore_id=g0), jnp.int32)
        iv = jnp.full((16,), plsc.fetch_and_add(smv.at[1], 0, subcore_id=g0), jnp.int32)
        for r in (1, 2, 3):
            kr = plsc.fetch_and_add(smv.at[0], 0, subcore_id=g0 + r)
            ir = plsc.fetch_and_add(smv.at[1], 0, subcore_id=g0 + r)
            kv = jnp.where(lane == r, jnp.full((16,), kr, jnp.int32), kv)
            iv = jnp.where(lane == r, jnp.full((16,), ir, jnp.int32), iv)
        bb = plsc.bitcast(kv, jnp.float32)
        m = allmax_f(bb)
        isp = allmin_i(jnp.where(bb == m, iv, jnp.int32(0x7FFFFFFF)))
        vsp = m > NEG_INF
        x1g = plsc.load_gather(x1v, [isp])
        y1g = plsc.load_gather(y1v, [isp])
        x2g = plsc.load_gather(x2v, [isp])
        y2g = plsc.load_gather(y2v, [isp])
        arg_ = plsc.load_gather(arv, [isp])
        x1s = jnp.where(vsp, x1g, BIGC)
        y1s = jnp.where(vsp, y1g, BIGC)
        x2s = jnp.where(vsp, x2g, -BIGC)
        y2s = jnp.where(vsp, y2g, -BIGC)
        t2sel = jnp.where(vsp, arg_, 0.0) * T2
        plsc.store_scatter(wkv, [isp], ninf)

        unroll = 4
        step = 16 * unroll

        def sweep_chunk(c, carry2):
            b2, bi2 = carry2
            base = c * step
            for u in range(unroll):
                off = qbase + base + u * 16
                sl = pl.ds(off, 16)
                iw = jnp.maximum(
                    jnp.minimum(x2s, x2v[sl]) - jnp.maximum(x1s, x1v[sl]),
                    0.0)
                ih = jnp.maximum(
                    jnp.minimum(y2s, y2v[sl]) - jnp.maximum(y1s, y1v[sl]),
                    0.0)
                inter = iw * ih
                kill = inter > t2sel + t2v[sl]
                wc = jnp.where(kill, NEG_INF, wkv[sl])
                wkv[sl] = wc
                gidx = lane + off
                upd = wc > b2
                b2 = jnp.where(upd, wc, b2)
                bi2 = jnp.where(upd, gidx, bi2)
            return b2, bi2

        best2, bidx2 = lax.fori_loop(0, QCHUNK // unroll, sweep_chunk,
                                     (ninf, zeroi))

        @pl.when(q == 0)
        def _():
            d0s = plsc.load_gather(d0v, [isp])
            d1s = plsc.load_gather(d1v, [isp])
            d2s = plsc.load_gather(d2v, [isp])
            d3s = plsc.load_gather(d3v, [isp])
            row = jnp.zeros((16,), jnp.float32)
            vals = (x1g, y1g, x2g, y2g, m, d0s, d1s, d2s, d3s)
            for j, vv in enumerate(vals):
                row = jnp.where(lane == j, vv, row)
            row = jnp.where(vsp, row, 0.0)
            plsc.store_scatter(outv, [jnp.full((16,), d, jnp.int32), lane],
                               row)

        plsc.subcore_barrier()
        return best2, bidx2

    lax.fori_loop(0, MAX_DET, det_step, (best, bidx))

    @pl.when(q == 0)
    def _():
        pltpu.sync_copy(outv, outh.at[b])


def _nms(rows, dist_t, batch):
    mesh = plsc.VectorSubcoreMesh(core_axis_name="c", subcore_axis_name="s")
    vec = pltpu.VMEM((N,), jnp.float32)
    f = pl.kernel(
        _nms_body,
        out_type=jax.ShapeDtypeStruct((batch, 112, 16), jnp.float32),
        mesh=mesh,
        compiler_params=pltpu.CompilerParams(needs_layout_passes=False),
        scratch_types=[vec] * 10 + [
            pltpu.VMEM((112, 16), jnp.float32),
            pltpu.VMEM((16,), jnp.float32),
            pltpu.VMEM((16,), jnp.int32),
            pltpu.VMEM((16,), jnp.int32),
            vec,
            pltpu.SMEM((8,), jnp.int32),
        ],
    )
    return f(*rows, dist_t)


def kernel(boxes, classes, distances, images):
    del images
    batch = boxes.shape[0]
    meta = _anchor_meta()
    boxes_t = jnp.transpose(boxes, (0, 2, 1))
    classes_t = jnp.transpose(classes, (0, 2, 1))
    dist_t = jnp.transpose(distances, (0, 2, 1))
    rows = _decode(boxes_t, classes_t, meta, batch)
    out = _nms(rows, dist_t, batch)
    return out[:, :MAX_DET, :9]


# speedup vs baseline: 1.0880x; 1.0580x over previous
import functools

import jax
import jax.numpy as jnp
import numpy as np
from jax import lax
from jax.experimental import pallas as pl
from jax.experimental.pallas import tpu as pltpu
from jax.experimental.pallas import tpu_sc as plsc

CONF_T = 0.2
IOU_T = 0.7
MAX_DET = 100
STRIDES = (8, 16, 32)
IMG_H, IMG_W = 512, 512
NUM_CLASSES = 80
N = sum((IMG_H // s) * (IMG_W // s) for s in STRIDES)
LANES = 16
NCHUNK = N // LANES
QUARTER = N // 4
QCHUNK = QUARTER // LANES
NEG_INF = float("-inf")
BIGC = 1.0e30
T2 = IOU_T / (1.0 + IOU_T)


def _anchor_meta():
    ax_l, ay_l, st_l = [], [], []
    for s in STRIDES:
        hh = np.arange(0, IMG_H, s, dtype=np.float32)
        ww = np.arange(0, IMG_W, s, dtype=np.float32)
        ww_g, hh_g = np.meshgrid(ww, hh)
        ay = (hh_g.reshape(-1) + 0.5 * s) / s
        ax = (ww_g.reshape(-1) + 0.5 * s) / s
        ax_l.append(ax)
        ay_l.append(ay)
        st_l.append(np.full(ax.shape, s, dtype=np.float32))
    meta = np.zeros((8, N), dtype=np.float32)
    meta[0] = np.concatenate(ax_l)
    meta[1] = np.concatenate(ay_l)
    meta[2] = np.concatenate(st_l)
    return jnp.asarray(meta)


def _decode_body(boxes_ref, classes_ref, meta_ref, x1_ref, y1_ref, x2_ref,
                 y2_ref, ar_ref, wk_ref):
    x = boxes_ref[0]
    db = []
    kcol = lax.broadcasted_iota(jnp.int32, (16, 1), 0).astype(jnp.float32)
    for s in range(4):
        xs = x[16 * s:16 * s + 16, :]
        m = jnp.max(xs, axis=0, keepdims=True)
        e = jnp.exp(xs - m)
        den = jnp.sum(e, axis=0, keepdims=True)
        num = jnp.sum(e * kcol, axis=0, keepdims=True)
        db.append(num / den)
    ax = meta_ref[0:1, :]
    ay = meta_ref[1:2, :]
    st = meta_ref[2:3, :]
    x1 = (ax - db[0]) * st
    y1 = (ay - db[1]) * st
    x2 = (ax + db[2]) * st
    y2 = (ay + db[3]) * st
    x1_ref[0] = x1
    y1_ref[0] = y1
    x2_ref[0] = x2
    y2_ref[0] = y2
    ar_ref[0] = jnp.maximum(x2 - x1, 0.0) * jnp.maximum(y2 - y1, 0.0)
    conf = jnp.max(classes_ref[0], axis=0, keepdims=True)
    wk_ref[0] = jnp.where(conf > CONF_T, conf, NEG_INF)


def _decode(boxes_t, classes_t, meta, batch):
    row = jax.ShapeDtypeStruct((batch, 1, N), jnp.float32)
    return pl.pallas_call(
        _decode_body,
        grid=(batch,),
        in_specs=[
            pl.BlockSpec((1, 64, N), lambda b: (b, 0, 0)),
            pl.BlockSpec((1, NUM_CLASSES, N), lambda b: (b, 0, 0)),
            pl.BlockSpec((8, N), lambda b: (0, 0)),
        ],
        out_specs=[pl.BlockSpec((1, 1, N), lambda b: (b, 0, 0))] * 6,
        out_shape=[row] * 6,
    )(boxes_t, classes_t, meta)


def _nms_body(x1h, y1h, x2h, y2h, arh, wkh, dsh, outh,
              x1v, y1v, x2v, y2v, arv, wkv, d0v, d1v, d2v, d3v, outv,
              scrf, scri, scrj, t2v, smv):
    sub = lax.axis_index("s")
    core = lax.axis_index("c")
    b = core * 4 + sub // 4
    q = sub % 4
    qbase = q * QUARTER
    g0 = (sub // 4) * 4

    pltpu.sync_copy(x1h.at[b, 0], x1v)
    pltpu.sync_copy(y1h.at[b, 0], y1v)
    pltpu.sync_copy(x2h.at[b, 0], x2v)
    pltpu.sync_copy(y2h.at[b, 0], y2v)
    pltpu.sync_copy(arh.at[b, 0], arv)
    pltpu.sync_copy(wkh.at[b, 0], wkv)

    @pl.when(q == 0)
    def _():
        pltpu.sync_copy(dsh.at[b, 0], d0v)
        pltpu.sync_copy(dsh.at[b, 1], d1v)
        pltpu.sync_copy(dsh.at[b, 2], d2v)
        pltpu.sync_copy(dsh.at[b, 3], d3v)

    lane = lax.iota(jnp.int32, 16)
    ninf = jnp.full((16,), NEG_INF, jnp.float32)
    zeroi = jnp.zeros((16,), jnp.int32)

    def init_chunk(c, carry):
        best, bidx = carry
        v = wkv[pl.ds(qbase + c * 16, 16)]
        gidx = lane + (qbase + c * 16)
        upd = v > best
        return jnp.where(upd, v, best), jnp.where(upd, gidx, bidx)

    best, bidx = lax.fori_loop(0, QCHUNK, init_chunk, (ninf, zeroi))

    def t2_chunk(c, carry):
        sl = pl.ds(qbase + c * 16, 16)
        t2v[sl] = arv[sl] * T2
        return carry

    lax.fori_loop(0, QCHUNK, t2_chunk, 0)

    def allmax_f(v):
        for k in (8, 4, 2, 1):
            scrf[...] = v
            v = jnp.maximum(v, plsc.load_gather(scrf, [lane ^ k]))
        return v

    def allmin_i(v):
        for k in (8, 4, 2, 1):
            scri[...] = v
            v = jnp.minimum(v, plsc.load_gather(scri, [lane ^ k]))
        return v

    def det_step(d, carry):
        best, bidx = carry
        mloc = allmax_f(best)
        iloc = allmin_i(jnp.where(best == mloc, bidx, jnp.int32(0x7FFFFFFF)))
        kb = plsc.bitcast(mloc, jnp.int32)[0]
        il = iloc[0]
        sbase = 2 * (d & 1)
        smv[sbase] = kb
        smv[sbase + 1] = il
        plsc.subcore_barrier()
        kv = jnp.full((16,),
                      plsc.fetch_and_add(smv.at[sbase], 0, subcore_id=g0),
                      jnp.int32)
        for r in (1, 2, 3):
            kr = plsc.fetch_and_add(smv.at[sbase], 0, subcore_id=g0 + r)
            kv = jnp.where(lane == r, jnp.full((16,), kr, jnp.int32), kv)
        bb = plsc.bitcast(kv, jnp.float32)
        m = allmax_f(bb)
        rwin = allmin_i(jnp.where(bb == m, lane, jnp.int32(0x7FFFFFFF)))[0]
        iwin = plsc.fetch_and_add(smv.at[sbase + 1], 0, subcore_id=g0 + rwin)
        isp = jnp.full((16,), iwin, jnp.int32)
        vsp = m > NEG_INF
        x1g = plsc.load_gather(x1v, [isp])
        y1g = plsc.load_gather(y1v, [isp])
        x2g = plsc.load_gather(x2v, [isp])
        y2g = plsc.load_gather(y2v, [isp])
        arg_ = plsc.load_gather(arv, [isp])
        x1s = jnp.where(vsp, x1g, BIGC)
        y1s = jnp.where(vsp, y1g, BIGC)
        x2s = jnp.where(vsp, x2g, -BIGC)
        y2s = jnp.where(vsp, y2g, -BIGC)
        t2sel = jnp.where(vsp, arg_, 0.0) * T2
        plsc.store_scatter(wkv, [isp], ninf)

        unroll = 4
        step = 16 * unroll

        def sweep_chunk(c, carry2):
            b2, bi2 = carry2
            base = c * step
            for u in range(unroll):
                off = qbase + base + u * 16
                sl = pl.ds(off, 16)
                iw = jnp.maximum(
                    jnp.minimum(x2s, x2v[sl]) - jnp.maximum(x1s, x1v[sl]),
                    0.0)
                ih = jnp.maximum(
                    jnp.minimum(y2s, y2v[sl]) - jnp.maximum(y1s, y1v[sl]),
                    0.0)
                inter = iw * ih
                kill = inter > t2sel + t2v[sl]
                wc = jnp.where(kill, NEG_INF, wkv[sl])
                wkv[sl] = wc
                gidx = lane + off
                upd = wc > b2
                b2 = jnp.where(upd, wc, b2)
                bi2 = jnp.where(upd, gidx, bi2)
            return b2, bi2

        best2, bidx2 = lax.fori_loop(0, QCHUNK // unroll, sweep_chunk,
                                     (ninf, zeroi))

        @pl.when(q == 0)
        def _():
            d0s = plsc.load_gather(d0v, [isp])
            d1s = plsc.load_gather(d1v, [isp])
            d2s = plsc.load_gather(d2v, [isp])
            d3s = plsc.load_gather(d3v, [isp])
            row = jnp.zeros((16,), jnp.float32)
            vals = (x1g, y1g, x2g, y2g, m, d0s, d1s, d2s, d3s)
            for j, vv in enumerate(vals):
                row = jnp.where(lane == j, vv, row)
            row = jnp.where(vsp, row, 0.0)
            plsc.store_scatter(outv, [jnp.full((16,), d, jnp.int32), lane],
                               row)

        return best2, bidx2

    lax.fori_loop(0, MAX_DET, det_step, (best, bidx))

    @pl.when(q == 0)
    def _():
        pltpu.sync_copy(outv, outh.at[b])


def _nms(rows, dist_t, batch):
    mesh = plsc.VectorSubcoreMesh(core_axis_name="c", subcore_axis_name="s")
    vec = pltpu.VMEM((N,), jnp.float32)
    f = pl.kernel(
        _nms_body,
        out_type=jax.ShapeDtypeStruct((batch, 112, 16), jnp.float32),
        mesh=mesh,
        compiler_params=pltpu.CompilerParams(needs_layout_passes=False),
        scratch_types=[vec] * 10 + [
            pltpu.VMEM((112, 16), jnp.float32),
            pltpu.VMEM((16,), jnp.float32),
            pltpu.VMEM((16,), jnp.int32),
            pltpu.VMEM((16,), jnp.int32),
            vec,
            pltpu.SMEM((8,), jnp.int32),
        ],
    )
    return f(*rows, dist_t)


def kernel(boxes, classes, distances, images):
    del images
    batch = boxes.shape[0]
    meta = _anchor_meta()
    boxes_t = jnp.transpose(boxes, (0, 2, 1))
    classes_t = jnp.transpose(classes, (0, 2, 1))
    dist_t = jnp.transpose(distances, (0, 2, 1))
    rows = _decode(boxes_t, classes_t, meta, batch)
    out = _nms(rows, dist_t, batch)
    return out[:, :MAX_DET, :9]


# speedup vs baseline: 1.1166x; 1.0263x over previous
import functools

import jax
import jax.numpy as jnp
import numpy as np
from jax import lax
from jax.experimental import pallas as pl
from jax.experimental.pallas import tpu as pltpu
from jax.experimental.pallas import tpu_sc as plsc

CONF_T = 0.2
IOU_T = 0.7
MAX_DET = 100
STRIDES = (8, 16, 32)
IMG_H, IMG_W = 512, 512
NUM_CLASSES = 80
N = sum((IMG_H // s) * (IMG_W // s) for s in STRIDES)
LANES = 16
NCHUNK = N // LANES
QUARTER = N // 4
QCHUNK = QUARTER // LANES
NEG_INF = float("-inf")
BIGC = 1.0e30
T2 = IOU_T / (1.0 + IOU_T)


def _anchor_meta():
    ax_l, ay_l, st_l = [], [], []
    for s in STRIDES:
        hh = np.arange(0, IMG_H, s, dtype=np.float32)
        ww = np.arange(0, IMG_W, s, dtype=np.float32)
        ww_g, hh_g = np.meshgrid(ww, hh)
        ay = (hh_g.reshape(-1) + 0.5 * s) / s
        ax = (ww_g.reshape(-1) + 0.5 * s) / s
        ax_l.append(ax)
        ay_l.append(ay)
        st_l.append(np.full(ax.shape, s, dtype=np.float32))
    meta = np.zeros((8, N), dtype=np.float32)
    meta[0] = np.concatenate(ax_l)
    meta[1] = np.concatenate(ay_l)
    meta[2] = np.concatenate(st_l)
    return jnp.asarray(meta)


def _decode_body(boxes_ref, classes_ref, meta_ref, x1_ref, y1_ref, x2_ref,
                 y2_ref, ar_ref, wk_ref):
    x = boxes_ref[0]
    db = []
    kcol = lax.broadcasted_iota(jnp.int32, (16, 1), 0).astype(jnp.float32)
    for s in range(4):
        xs = x[16 * s:16 * s + 16, :]
        m = jnp.max(xs, axis=0, keepdims=True)
        e = jnp.exp(xs - m)
        den = jnp.sum(e, axis=0, keepdims=True)
        num = jnp.sum(e * kcol, axis=0, keepdims=True)
        db.append(num / den)
    ax = meta_ref[0:1, :]
    ay = meta_ref[1:2, :]
    st = meta_ref[2:3, :]
    x1 = (ax - db[0]) * st
    y1 = (ay - db[1]) * st
    x2 = (ax + db[2]) * st
    y2 = (ay + db[3]) * st
    x1_ref[0] = x1
    y1_ref[0] = y1
    x2_ref[0] = x2
    y2_ref[0] = y2
    ar_ref[0] = jnp.maximum(x2 - x1, 0.0) * jnp.maximum(y2 - y1, 0.0)
    conf = jnp.max(classes_ref[0], axis=0, keepdims=True)
    wk_ref[0] = jnp.where(conf > CONF_T, conf, NEG_INF)


def _decode(boxes_t, classes_t, meta, batch):
    row = jax.ShapeDtypeStruct((batch, 1, N), jnp.float32)
    return pl.pallas_call(
        _decode_body,
        grid=(batch,),
        in_specs=[
            pl.BlockSpec((1, 64, N), lambda b: (b, 0, 0)),
            pl.BlockSpec((1, NUM_CLASSES, N), lambda b: (b, 0, 0)),
            pl.BlockSpec((8, N), lambda b: (0, 0)),
        ],
        out_specs=[pl.BlockSpec((1, 1, N), lambda b: (b, 0, 0))] * 6,
        out_shape=[row] * 6,
    )(boxes_t, classes_t, meta)


def _nms_body(x1h, y1h, x2h, y2h, arh, wkh, dsh, outh,
              x1v, y1v, x2v, y2v, arv, wkv, d0v, d1v, d2v, d3v, outv,
              scrf, scri, scrj, t2v, smv):
    sub = lax.axis_index("s")
    core = lax.axis_index("c")
    b = core * 4 + sub // 4
    q = sub % 4
    qbase = q * QUARTER
    g0 = (sub // 4) * 4

    pltpu.sync_copy(x1h.at[b, 0], x1v)
    pltpu.sync_copy(y1h.at[b, 0], y1v)
    pltpu.sync_copy(x2h.at[b, 0], x2v)
    pltpu.sync_copy(y2h.at[b, 0], y2v)
    pltpu.sync_copy(arh.at[b, 0], arv)
    pltpu.sync_copy(wkh.at[b, 0], wkv)

    @pl.when(q == 0)
    def _():
        pltpu.sync_copy(dsh.at[b, 0], d0v)
        pltpu.sync_copy(dsh.at[b, 1], d1v)
        pltpu.sync_copy(dsh.at[b, 2], d2v)
        pltpu.sync_copy(dsh.at[b, 3], d3v)

    lane = lax.iota(jnp.int32, 16)
    ninf = jnp.full((16,), NEG_INF, jnp.float32)
    zeroi = jnp.zeros((16,), jnp.int32)

    def init_chunk(c, carry):
        best, bidx = carry
        v = wkv[pl.ds(qbase + c * 16, 16)]
        gidx = lane + (qbase + c * 16)
        upd = v > best
        return jnp.where(upd, v, best), jnp.where(upd, gidx, bidx)

    best, bidx = lax.fori_loop(0, QCHUNK, init_chunk, (ninf, zeroi))

    def t2_chunk(c, carry):
        sl = pl.ds(qbase + c * 16, 16)
        t2v[sl] = arv[sl] * T2
        return carry

    lax.fori_loop(0, QCHUNK, t2_chunk, 0)

    def allmax_f(v):
        for k in (8, 4, 2, 1):
            v = jnp.maximum(v, v.at[lane ^ k].get(mode="promise_in_bounds"))
        return v

    def allmin_i(v):
        for k in (8, 4, 2, 1):
            v = jnp.minimum(v, v.at[lane ^ k].get(mode="promise_in_bounds"))
        return v

    def det_step(d, carry):
        best, bidx = carry
        mloc = allmax_f(best)
        iloc = allmin_i(jnp.where(best == mloc, bidx, jnp.int32(0x7FFFFFFF)))
        kb = plsc.bitcast(mloc, jnp.int32)[0]
        il = iloc[0]
        sbase = 2 * (d & 1)
        smv[sbase] = kb
        smv[sbase + 1] = il
        plsc.subcore_barrier()
        kv = jnp.full((16,),
                      plsc.fetch_and_add(smv.at[sbase], 0, subcore_id=g0),
                      jnp.int32)
        for r in (1, 2, 3):
            kr = plsc.fetch_and_add(smv.at[sbase], 0, subcore_id=g0 + r)
            kv = jnp.where(lane == r, jnp.full((16,), kr, jnp.int32), kv)
        bb = plsc.bitcast(kv, jnp.float32)
        m = allmax_f(bb)
        rwin = allmin_i(jnp.where(bb == m, lane, jnp.int32(0x7FFFFFFF)))[0]
        iwin = plsc.fetch_and_add(smv.at[sbase + 1], 0, subcore_id=g0 + rwin)
        isp = jnp.full((16,), iwin, jnp.int32)
        vsp = m > NEG_INF
        x1g = plsc.load_gather(x1v, [isp])
        y1g = plsc.load_gather(y1v, [isp])
        x2g = plsc.load_gather(x2v, [isp])
        y2g = plsc.load_gather(y2v, [isp])
        arg_ = plsc.load_gather(arv, [isp])
        x1s = jnp.where(vsp, x1g, BIGC)
        y1s = jnp.where(vsp, y1g, BIGC)
        x2s = jnp.where(vsp, x2g, -BIGC)
        y2s = jnp.where(vsp, y2g, -BIGC)
        t2sel = jnp.where(vsp, arg_, 0.0) * T2
        plsc.store_scatter(wkv, [isp], ninf)

        unroll = 4
        step = 16 * unroll

        def sweep_chunk(c, carry2):
            b2, bi2 = carry2
            base = c * step
            for u in range(unroll):
                off = qbase + base + u * 16
                sl = pl.ds(off, 16)
                iw = jnp.maximum(
                    jnp.minimum(x2s, x2v[sl]) - jnp.maximum(x1s, x1v[sl]),
                    0.0)
                ih = jnp.maximum(
                    jnp.minimum(y2s, y2v[sl]) - jnp.maximum(y1s, y1v[sl]),
                    0.0)
                inter = iw * ih
                kill = inter > t2sel + t2v[sl]
                wc = jnp.where(kill, NEG_INF, wkv[sl])
                wkv[sl] = wc
                gidx = lane + off
                upd = wc > b2
                b2 = jnp.where(upd, wc, b2)
                bi2 = jnp.where(upd, gidx, bi2)
            return b2, bi2

        best2, bidx2 = lax.fori_loop(0, QCHUNK // unroll, sweep_chunk,
                                     (ninf, zeroi))

        @pl.when(q == 0)
        def _():
            d0s = plsc.load_gather(d0v, [isp])
            d1s = plsc.load_gather(d1v, [isp])
            d2s = plsc.load_gather(d2v, [isp])
            d3s = plsc.load_gather(d3v, [isp])
            row = jnp.zeros((16,), jnp.float32)
            vals = (x1g, y1g, x2g, y2g, m, d0s, d1s, d2s, d3s)
            for j, vv in enumerate(vals):
                row = jnp.where(lane == j, vv, row)
            row = jnp.where(vsp, row, 0.0)
            plsc.store_scatter(outv, [jnp.full((16,), d, jnp.int32), lane],
                               row)

        return best2, bidx2

    lax.fori_loop(0, MAX_DET, det_step, (best, bidx))

    @pl.when(q == 0)
    def _():
        pltpu.sync_copy(outv, outh.at[b])


def _nms(rows, dist_t, batch):
    mesh = plsc.VectorSubcoreMesh(core_axis_name="c", subcore_axis_name="s")
    vec = pltpu.VMEM((N,), jnp.float32)
    f = pl.kernel(
        _nms_body,
        out_type=jax.ShapeDtypeStruct((batch, 112, 16), jnp.float32),
        mesh=mesh,
        compiler_params=pltpu.CompilerParams(needs_layout_passes=False),
        scratch_types=[vec] * 10 + [
            pltpu.VMEM((112, 16), jnp.float32),
            pltpu.VMEM((16,), jnp.float32),
            pltpu.VMEM((16,), jnp.int32),
            pltpu.VMEM((16,), jnp.int32),
            vec,
            pltpu.SMEM((8,), jnp.int32),
        ],
    )
    return f(*rows, dist_t)


def kernel(boxes, classes, distances, images):
    del images
    batch = boxes.shape[0]
    meta = _anchor_meta()
    boxes_t = jnp.transpose(boxes, (0, 2, 1))
    classes_t = jnp.transpose(classes, (0, 2, 1))
    dist_t = jnp.transpose(distances, (0, 2, 1))
    rows = _decode(boxes_t, classes_t, meta, batch)
    out = _nms(rows, dist_t, batch)
    return out[:, :MAX_DET, :9]


# speedup vs baseline: 1.1186x; 1.0018x over previous
import functools

import jax
import jax.numpy as jnp
import numpy as np
from jax import lax
from jax.experimental import pallas as pl
from jax.experimental.pallas import tpu as pltpu
from jax.experimental.pallas import tpu_sc as plsc

CONF_T = 0.2
IOU_T = 0.7
MAX_DET = 100
STRIDES = (8, 16, 32)
IMG_H, IMG_W = 512, 512
NUM_CLASSES = 80
N = sum((IMG_H // s) * (IMG_W // s) for s in STRIDES)
LANES = 16
NCHUNK = N // LANES
QUARTER = N // 4
QCHUNK = QUARTER // LANES
NEG_INF = float("-inf")
BIGC = 1.0e30
T2 = IOU_T / (1.0 + IOU_T)


def _anchor_meta():
    ax_l, ay_l, st_l = [], [], []
    for s in STRIDES:
        hh = np.arange(0, IMG_H, s, dtype=np.float32)
        ww = np.arange(0, IMG_W, s, dtype=np.float32)
        ww_g, hh_g = np.meshgrid(ww, hh)
        ay = (hh_g.reshape(-1) + 0.5 * s) / s
        ax = (ww_g.reshape(-1) + 0.5 * s) / s
        ax_l.append(ax)
        ay_l.append(ay)
        st_l.append(np.full(ax.shape, s, dtype=np.float32))
    meta = np.zeros((8, N), dtype=np.float32)
    meta[0] = np.concatenate(ax_l)
    meta[1] = np.concatenate(ay_l)
    meta[2] = np.concatenate(st_l)
    return jnp.asarray(meta)


def _decode_body(boxes_ref, classes_ref, meta_ref, x1_ref, y1_ref, x2_ref,
                 y2_ref, ar_ref, wk_ref):
    x = boxes_ref[0]
    db = []
    kcol = lax.broadcasted_iota(jnp.int32, (16, 1), 0).astype(jnp.float32)
    for s in range(4):
        xs = x[16 * s:16 * s + 16, :]
        m = jnp.max(xs, axis=0, keepdims=True)
        e = jnp.exp(xs - m)
        den = jnp.sum(e, axis=0, keepdims=True)
        num = jnp.sum(e * kcol, axis=0, keepdims=True)
        db.append(num / den)
    ax = meta_ref[0:1, :]
    ay = meta_ref[1:2, :]
    st = meta_ref[2:3, :]
    x1 = (ax - db[0]) * st
    y1 = (ay - db[1]) * st
    x2 = (ax + db[2]) * st
    y2 = (ay + db[3]) * st
    x1_ref[0] = x1
    y1_ref[0] = y1
    x2_ref[0] = x2
    y2_ref[0] = y2
    ar_ref[0] = jnp.maximum(x2 - x1, 0.0) * jnp.maximum(y2 - y1, 0.0)
    conf = jnp.max(classes_ref[0], axis=0, keepdims=True)
    wk_ref[0] = jnp.where(conf > CONF_T, conf, NEG_INF)


def _decode(boxes_t, classes_t, meta, batch):
    row = jax.ShapeDtypeStruct((batch, 1, N), jnp.float32)
    return pl.pallas_call(
        _decode_body,
        grid=(batch,),
        in_specs=[
            pl.BlockSpec((1, 64, N), lambda b: (b, 0, 0)),
            pl.BlockSpec((1, NUM_CLASSES, N), lambda b: (b, 0, 0)),
            pl.BlockSpec((8, N), lambda b: (0, 0)),
        ],
        out_specs=[pl.BlockSpec((1, 1, N), lambda b: (b, 0, 0))] * 6,
        out_shape=[row] * 6,
    )(boxes_t, classes_t, meta)


def _nms_body(x1h, y1h, x2h, y2h, arh, wkh, dsh, outh,
              x1v, y1v, x2v, y2v, arv, wkv, d0v, d1v, d2v, d3v, outv,
              scrf, scri, scrj, t2v, smv):
    sub = lax.axis_index("s")
    core = lax.axis_index("c")
    b = core * 4 + sub // 4
    q = sub % 4
    qbase = q * QUARTER
    g0 = (sub // 4) * 4

    pltpu.sync_copy(x1h.at[b, 0], x1v)
    pltpu.sync_copy(y1h.at[b, 0], y1v)
    pltpu.sync_copy(x2h.at[b, 0], x2v)
    pltpu.sync_copy(y2h.at[b, 0], y2v)
    pltpu.sync_copy(arh.at[b, 0], arv)
    pltpu.sync_copy(wkh.at[b, 0], wkv)

    @pl.when(q == 0)
    def _():
        pltpu.sync_copy(dsh.at[b, 0], d0v)
        pltpu.sync_copy(dsh.at[b, 1], d1v)
        pltpu.sync_copy(dsh.at[b, 2], d2v)
        pltpu.sync_copy(dsh.at[b, 3], d3v)

    lane = lax.iota(jnp.int32, 16)
    ninf = jnp.full((16,), NEG_INF, jnp.float32)
    zeroi = jnp.zeros((16,), jnp.int32)

    def init_chunk(c, carry):
        best, bidx = carry
        v = wkv[pl.ds(qbase + c * 16, 16)]
        gidx = lane + (qbase + c * 16)
        upd = v > best
        return jnp.where(upd, v, best), jnp.where(upd, gidx, bidx)

    best, bidx = lax.fori_loop(0, QCHUNK, init_chunk, (ninf, zeroi))

    def t2_chunk(c, carry):
        sl = pl.ds(qbase + c * 16, 16)
        t2v[sl] = arv[sl] * T2
        return carry

    lax.fori_loop(0, QCHUNK, t2_chunk, 0)

    def allmax_f(v):
        for k in (8, 4, 2, 1):
            v = jnp.maximum(v, v.at[lane ^ k].get(mode="promise_in_bounds"))
        return v

    def allmin_i(v):
        for k in (8, 4, 2, 1):
            v = jnp.minimum(v, v.at[lane ^ k].get(mode="promise_in_bounds"))
        return v

    def det_step(d, carry):
        best, bidx = carry
        mloc = allmax_f(best)
        iloc = allmin_i(jnp.where(best == mloc, bidx, jnp.int32(0x7FFFFFFF)))
        kb = plsc.bitcast(mloc, jnp.int32)[0]
        il = iloc[0]
        sbase = 2 * (d & 1)
        smv[sbase] = kb
        smv[sbase + 1] = il
        plsc.subcore_barrier()
        kv = jnp.full((16,),
                      plsc.fetch_and_add(smv.at[sbase], 0, subcore_id=g0),
                      jnp.int32)
        for r in (1, 2, 3):
            kr = plsc.fetch_and_add(smv.at[sbase], 0, subcore_id=g0 + r)
            kv = jnp.where(lane == r, jnp.full((16,), kr, jnp.int32), kv)
        bb = plsc.bitcast(kv, jnp.float32)
        m = allmax_f(bb)
        rwin = allmin_i(jnp.where(bb == m, lane, jnp.int32(0x7FFFFFFF)))[0]
        iwin = plsc.fetch_and_add(smv.at[sbase + 1], 0, subcore_id=g0 + rwin)
        isp = jnp.full((16,), iwin, jnp.int32)
        vsp = m > NEG_INF
        x1g = plsc.load_gather(x1v, [isp])
        y1g = plsc.load_gather(y1v, [isp])
        x2g = plsc.load_gather(x2v, [isp])
        y2g = plsc.load_gather(y2v, [isp])
        arg_ = plsc.load_gather(arv, [isp])
        x1s = jnp.where(vsp, x1g, BIGC)
        y1s = jnp.where(vsp, y1g, BIGC)
        x2s = jnp.where(vsp, x2g, -BIGC)
        y2s = jnp.where(vsp, y2g, -BIGC)
        t2sel = jnp.where(vsp, arg_, 0.0) * T2
        plsc.store_scatter(wkv, [isp], ninf)

        unroll = 6
        step = 16 * unroll

        def sweep_chunk(c, carry2):
            b2, bi2 = carry2
            base = c * step
            for u in range(unroll):
                off = qbase + base + u * 16
                sl = pl.ds(off, 16)
                iw = jnp.maximum(
                    jnp.minimum(x2s, x2v[sl]) - jnp.maximum(x1s, x1v[sl]),
                    0.0)
                ih = jnp.maximum(
                    jnp.minimum(y2s, y2v[sl]) - jnp.maximum(y1s, y1v[sl]),
                    0.0)
                inter = iw * ih
                kill = inter > t2sel + t2v[sl]
                wc = jnp.where(kill, NEG_INF, wkv[sl])
                wkv[sl] = wc
                gidx = lane + off
                upd = wc > b2
                b2 = jnp.where(upd, wc, b2)
                bi2 = jnp.where(upd, gidx, bi2)
            return b2, bi2

        best2, bidx2 = lax.fori_loop(0, QCHUNK // unroll, sweep_chunk,
                                     (ninf, zeroi))

        @pl.when(q == 0)
        def _():
            d0s = plsc.load_gather(d0v, [isp])
            d1s = plsc.load_gather(d1v, [isp])
            d2s = plsc.load_gather(d2v, [isp])
            d3s = plsc.load_gather(d3v, [isp])
            row = jnp.zeros((16,), jnp.float32)
            vals = (x1g, y1g, x2g, y2g, m, d0s, d1s, d2s, d3s)
            for j, vv in enumerate(vals):
                row = jnp.where(lane == j, vv, row)
            row = jnp.where(vsp, row, 0.0)
            plsc.store_scatter(outv, [jnp.full((16,), d, jnp.int32), lane],
                               row)

        return best2, bidx2

    lax.fori_loop(0, MAX_DET, det_step, (best, bidx))

    @pl.when(q == 0)
    def _():
        pltpu.sync_copy(outv, outh.at[b])


def _nms(rows, dist_t, batch):
    mesh = plsc.VectorSubcoreMesh(core_axis_name="c", subcore_axis_name="s")
    vec = pltpu.VMEM((N,), jnp.float32)
    f = pl.kernel(
        _nms_body,
        out_type=jax.ShapeDtypeStruct((batch, 112, 16), jnp.float32),
        mesh=mesh,
        compiler_params=pltpu.CompilerParams(needs_layout_passes=False),
        scratch_types=[vec] * 10 + [
            pltpu.VMEM((112, 16), jnp.float32),
            pltpu.VMEM((16,), jnp.float32),
            pltpu.VMEM((16,), jnp.int32),
            pltpu.VMEM((16,), jnp.int32),
            vec,
            pltpu.SMEM((8,), jnp.int32),
        ],
    )
    return f(*rows, dist_t)


def kernel(boxes, classes, distances, images):
    del images
    batch = boxes.shape[0]
    meta = _anchor_meta()
    boxes_t = jnp.transpose(boxes, (0, 2, 1))
    classes_t = jnp.transpose(classes, (0, 2, 1))
    dist_t = jnp.transpose(distances, (0, 2, 1))
    rows = _decode(boxes_t, classes_t, meta, batch)
    out = _nms(rows, dist_t, batch)
    return out[:, :MAX_DET, :9]
